# baseline (speedup 1.0000x reference)
import jax
import jax.numpy as jnp
from jax.experimental import pallas as pl


def _copy_body(x_ref, o_ref):
    o_ref[...] = x_ref[...]


def kernel(logits):
    B, T, V = logits.shape
    x = logits.reshape(B * T, V)
    rows = B * T
    bt = min(256, rows)
    bv = min(6400, V)
    out = pl.pallas_call(
        _copy_body,
        grid=(pl.cdiv(rows, bt), pl.cdiv(V, bv)),
        in_specs=[pl.BlockSpec((bt, bv), lambda i, j: (i, j))],
        out_specs=pl.BlockSpec((bt, bv), lambda i, j: (i, j)),
        out_shape=jax.ShapeDtypeStruct((rows, V), logits.dtype),
    )(x)
    return out.reshape(B, T, V)
